# Initial kernel scaffold; baseline (speedup 1.0000x reference)
import jax
import jax.numpy as jnp
from jax import lax
from jax.experimental import pallas as pl
from jax.experimental.pallas import tpu as pltpu
from jax.experimental.pallas import tpu_sc as plsc

_MAX_MZ = 1500.0
_NUM_BINS = 1500
_B = 1024
_T = 200

_NB_PAD = 1504
_T_PAD = 208
_NW = 32
_ROWS = _B // _NW
_LANES = 16


def _sc_body(pred_hbm, mz_hbm, it_hbm, mk_hbm,
             dot_hbm, pn2_hbm, tn2_hbm,
             pred_v, mz_v, it_v, mk_v, hist_v, dot_v, pn2_v, tn2_v):
  c = lax.axis_index("c")
  s = lax.axis_index("s")
  wid = s * 2 + c
  base = wid * _ROWS

  pltpu.sync_copy(pred_hbm.at[pl.ds(base, _ROWS)], pred_v)
  pltpu.sync_copy(mz_hbm.at[pl.ds(base, _ROWS)], mz_v)
  pltpu.sync_copy(it_hbm.at[pl.ds(base, _ROWS)], it_v)
  pltpu.sync_copy(mk_hbm.at[pl.ds(base, _ROWS)], mk_v)

  zeros = jnp.zeros((_LANES,), jnp.float32)

  def zero_hist(j, carry):
    hist_v[pl.ds(j * _LANES, _LANES)] = zeros
    return carry

  lax.fori_loop(0, _NB_PAD // _LANES, zero_hist, 0)

  def row_body(r, carry):
    rsplat = jnp.full((_LANES,), r, jnp.int32)

    def pn2_body(j, acc):
      v = pred_v[r, pl.ds(j * _LANES, _LANES)]
      return acc + v * v

    pn2 = jnp.sum(lax.fori_loop(0, _NB_PAD // _LANES, pn2_body,
                                jnp.zeros((_LANES,), jnp.float32)))

    def dot_body(j, acc):
      mz = mz_v[r, pl.ds(j * _LANES, _LANES)]
      w = it_v[r, pl.ds(j * _LANES, _LANES)] * mk_v[r, pl.ds(j * _LANES, _LANES)]
      idx = jnp.clip((mz * _MAX_MZ).astype(jnp.int32), 0, _NUM_BINS - 1)
      g = plsc.load_gather(pred_v, [rsplat, idx])
      plsc.addupdate_scatter(hist_v, [idx], w)
      return acc + g * w

    dot = jnp.sum(lax.fori_loop(0, _T_PAD // _LANES, dot_body,
                                jnp.zeros((_LANES,), jnp.float32)))

    def tn2_body(j, acc):
      mz = mz_v[r, pl.ds(j * _LANES, _LANES)]
      w = it_v[r, pl.ds(j * _LANES, _LANES)] * mk_v[r, pl.ds(j * _LANES, _LANES)]
      idx = jnp.clip((mz * _MAX_MZ).astype(jnp.int32), 0, _NUM_BINS - 1)
      hv = plsc.load_gather(hist_v, [idx])
      return acc + hv * w

    tn2 = jnp.sum(lax.fori_loop(0, _T_PAD // _LANES, tn2_body,
                                jnp.zeros((_LANES,), jnp.float32)))

    def reset_body(j, carry2):
      mz = mz_v[r, pl.ds(j * _LANES, _LANES)]
      idx = jnp.clip((mz * _MAX_MZ).astype(jnp.int32), 0, _NUM_BINS - 1)
      plsc.store_scatter(hist_v, [idx], zeros)
      return carry2

    lax.fori_loop(0, _T_PAD // _LANES, reset_body, 0)

    dot_v[r] = dot
    pn2_v[r] = pn2
    tn2_v[r] = tn2
    return carry

  lax.fori_loop(0, _ROWS, row_body, 0)

  pltpu.sync_copy(dot_v, dot_hbm.at[pl.ds(base, _ROWS)])
  pltpu.sync_copy(pn2_v, pn2_hbm.at[pl.ds(base, _ROWS)])
  pltpu.sync_copy(tn2_v, tn2_hbm.at[pl.ds(base, _ROWS)])


def _combine_body(dot_ref, pn2_ref, tn2_ref, out_ref):
  dot = dot_ref[...]
  pnorm = jnp.sqrt(pn2_ref[...])
  tnorm = jnp.sqrt(tn2_ref[...])
  num = dot / ((pnorm + 1e-8) * (tnorm + 1e-8))
  na = jnp.maximum(pnorm / (pnorm + 1e-8), 1e-8)
  nb = jnp.maximum(tnorm / (tnorm + 1e-8), 1e-8)
  cos = num / (na * nb)
  out_ref[0, 0] = 1.0 - jnp.mean(cos)


def kernel(pred_binned, target_mz, target_intensity, target_mask):
  pred_p = jnp.pad(pred_binned, ((0, 0), (0, _NB_PAD - _NUM_BINS)))
  mz_p = jnp.pad(target_mz, ((0, 0), (0, _T_PAD - _T)))
  it_p = jnp.pad(target_intensity, ((0, 0), (0, _T_PAD - _T)))
  mk_p = jnp.pad(target_mask, ((0, 0), (0, _T_PAD - _T)))

  sc = pl.kernel(
      _sc_body,
      mesh=plsc.VectorSubcoreMesh(core_axis_name="c", subcore_axis_name="s"),
      out_type=(
          jax.ShapeDtypeStruct((_B,), jnp.float32),
          jax.ShapeDtypeStruct((_B,), jnp.float32),
          jax.ShapeDtypeStruct((_B,), jnp.float32),
      ),
      scratch_types=[
          pltpu.VMEM((_ROWS, _NB_PAD), jnp.float32),
          pltpu.VMEM((_ROWS, _T_PAD), jnp.float32),
          pltpu.VMEM((_ROWS, _T_PAD), jnp.float32),
          pltpu.VMEM((_ROWS, _T_PAD), jnp.float32),
          pltpu.VMEM((_NB_PAD,), jnp.float32),
          pltpu.VMEM((_ROWS,), jnp.float32),
          pltpu.VMEM((_ROWS,), jnp.float32),
          pltpu.VMEM((_ROWS,), jnp.float32),
      ],
  )
  dot, pn2, tn2 = sc(pred_p, mz_p, it_p, mk_p)

  out = pl.pallas_call(
      _combine_body,
      out_shape=jax.ShapeDtypeStruct((1, 1), jnp.float32),
  )(dot.reshape(8, 128), pn2.reshape(8, 128), tn2.reshape(8, 128))
  return out.reshape(())


# speedup vs baseline: 7.2089x; 7.2089x over previous
import jax
import jax.numpy as jnp
from jax import lax
from jax.experimental import pallas as pl
from jax.experimental.pallas import tpu as pltpu
from jax.experimental.pallas import tpu_sc as plsc

_MAX_MZ = 1500.0
_NUM_BINS = 1500
_B = 1024
_T = 200

_NB_PAD = 1504
_T_PAD = 208
_NW = 32
_ROWS = _B // _NW
_LANES = 16


def _sc_body(pred_hbm, mz_hbm, it_hbm, mk_hbm,
             dot_hbm, pn2_hbm, tn2_hbm,
             pred_v, mz_v, it_v, mk_v, hist_v, dot_v, pn2_v, tn2_v):
  c = lax.axis_index("c")
  s = lax.axis_index("s")
  wid = s * 2 + c
  base = wid * _ROWS

  pltpu.sync_copy(pred_hbm.at[pl.ds(base * _NB_PAD, _ROWS * _NB_PAD)], pred_v)
  pltpu.sync_copy(mz_hbm.at[pl.ds(base * _T_PAD, _ROWS * _T_PAD)], mz_v)
  pltpu.sync_copy(it_hbm.at[pl.ds(base * _T_PAD, _ROWS * _T_PAD)], it_v)
  pltpu.sync_copy(mk_hbm.at[pl.ds(base * _T_PAD, _ROWS * _T_PAD)], mk_v)

  zeros = jnp.zeros((_LANES,), jnp.float32)

  def zero_hist(j, carry):
    hist_v[pl.ds(j * _LANES, _LANES)] = zeros
    return carry

  lax.fori_loop(0, _NB_PAD // _LANES, zero_hist, 0)

  def row_body(r, carry):
    pbase = r * _NB_PAD
    tbase = r * _T_PAD
    psplat = jnp.full((_LANES,), pbase, jnp.int32)

    def pn2_body(j, acc):
      v = pred_v[pl.ds(pbase + j * _LANES, _LANES)]
      return acc + v * v

    pn2 = lax.fori_loop(0, _NB_PAD // _LANES, pn2_body,
                        jnp.zeros((_LANES,), jnp.float32))

    def dot_body(j, acc):
      mz = mz_v[pl.ds(tbase + j * _LANES, _LANES)]
      w = (it_v[pl.ds(tbase + j * _LANES, _LANES)]
           * mk_v[pl.ds(tbase + j * _LANES, _LANES)])
      idx = jnp.clip((mz * _MAX_MZ).astype(jnp.int32), 0, _NUM_BINS - 1)
      g = plsc.load_gather(pred_v, [psplat + idx])
      plsc.addupdate_scatter(hist_v, [idx], w)
      return acc + g * w

    dot = lax.fori_loop(0, _T_PAD // _LANES, dot_body,
                        jnp.zeros((_LANES,), jnp.float32))

    def tn2_body(j, acc):
      mz = mz_v[pl.ds(tbase + j * _LANES, _LANES)]
      w = (it_v[pl.ds(tbase + j * _LANES, _LANES)]
           * mk_v[pl.ds(tbase + j * _LANES, _LANES)])
      idx = jnp.clip((mz * _MAX_MZ).astype(jnp.int32), 0, _NUM_BINS - 1)
      hv = plsc.load_gather(hist_v, [idx])
      return acc + hv * w

    tn2 = lax.fori_loop(0, _T_PAD // _LANES, tn2_body,
                        jnp.zeros((_LANES,), jnp.float32))

    def reset_body(j, carry2):
      mz = mz_v[pl.ds(tbase + j * _LANES, _LANES)]
      idx = jnp.clip((mz * _MAX_MZ).astype(jnp.int32), 0, _NUM_BINS - 1)
      plsc.store_scatter(hist_v, [idx], zeros)
      return carry2

    lax.fori_loop(0, _T_PAD // _LANES, reset_body, 0)

    dot_v[pl.ds(r * _LANES, _LANES)] = dot
    pn2_v[pl.ds(r * _LANES, _LANES)] = pn2
    tn2_v[pl.ds(r * _LANES, _LANES)] = tn2
    return carry

  lax.fori_loop(0, _ROWS, row_body, 0)

  pltpu.sync_copy(dot_v, dot_hbm.at[pl.ds(base * _LANES, _ROWS * _LANES)])
  pltpu.sync_copy(pn2_v, pn2_hbm.at[pl.ds(base * _LANES, _ROWS * _LANES)])
  pltpu.sync_copy(tn2_v, tn2_hbm.at[pl.ds(base * _LANES, _ROWS * _LANES)])


def _combine_body(dot_ref, pn2_ref, tn2_ref, out_ref):
  dot = jnp.sum(dot_ref[...], axis=1)
  pnorm = jnp.sqrt(jnp.sum(pn2_ref[...], axis=1))
  tnorm = jnp.sqrt(jnp.sum(tn2_ref[...], axis=1))
  num = dot / ((pnorm + 1e-8) * (tnorm + 1e-8))
  na = jnp.maximum(pnorm / (pnorm + 1e-8), 1e-8)
  nb = jnp.maximum(tnorm / (tnorm + 1e-8), 1e-8)
  cos = num / (na * nb)
  out_ref[...] = jnp.broadcast_to(1.0 - jnp.mean(cos), (1, 1))


def kernel(pred_binned, target_mz, target_intensity, target_mask):
  pred_p = jnp.pad(pred_binned, ((0, 0), (0, _NB_PAD - _NUM_BINS))).reshape(-1)
  mz_p = jnp.pad(target_mz, ((0, 0), (0, _T_PAD - _T))).reshape(-1)
  it_p = jnp.pad(target_intensity, ((0, 0), (0, _T_PAD - _T))).reshape(-1)
  mk_p = jnp.pad(target_mask, ((0, 0), (0, _T_PAD - _T))).reshape(-1)

  sc = pl.kernel(
      _sc_body,
      mesh=plsc.VectorSubcoreMesh(core_axis_name="c", subcore_axis_name="s"),
      compiler_params=pltpu.CompilerParams(
          use_tc_tiling_on_sc=False, needs_layout_passes=False),
      out_type=(
          jax.ShapeDtypeStruct((_B * _LANES,), jnp.float32),
          jax.ShapeDtypeStruct((_B * _LANES,), jnp.float32),
          jax.ShapeDtypeStruct((_B * _LANES,), jnp.float32),
      ),
      scratch_types=[
          pltpu.VMEM((_ROWS * _NB_PAD,), jnp.float32),
          pltpu.VMEM((_ROWS * _T_PAD,), jnp.float32),
          pltpu.VMEM((_ROWS * _T_PAD,), jnp.float32),
          pltpu.VMEM((_ROWS * _T_PAD,), jnp.float32),
          pltpu.VMEM((_NB_PAD,), jnp.float32),
          pltpu.VMEM((_ROWS * _LANES,), jnp.float32),
          pltpu.VMEM((_ROWS * _LANES,), jnp.float32),
          pltpu.VMEM((_ROWS * _LANES,), jnp.float32),
      ],
  )
  dot, pn2, tn2 = sc(pred_p, mz_p, it_p, mk_p)

  out = pl.pallas_call(
      _combine_body,
      out_shape=jax.ShapeDtypeStruct((1, 1), jnp.float32),
  )(dot.reshape(_B, _LANES), pn2.reshape(_B, _LANES), tn2.reshape(_B, _LANES))
  return out.reshape(())


# speedup vs baseline: 9.1253x; 1.2658x over previous
import jax
import jax.numpy as jnp
from jax import lax
from jax.experimental import pallas as pl
from jax.experimental.pallas import tpu as pltpu
from jax.experimental.pallas import tpu_sc as plsc

_MAX_MZ = 1500.0
_NUM_BINS = 1500
_B = 1024
_T = 200

_NB_PAD = 1504
_NW = 32
_ROWS = _B // _NW
_LANES = 16
_TG = 13
_T_TAIL = _T - (_TG - 1) * _LANES


def _sc_body(pred_hbm, mz_hbm, it_hbm, mk_hbm,
             dot_hbm, tn2_hbm,
             pred_v, mz_v, it_v, mk_v, hist_v, idx_v, w_v, dot_v, tn2_v):
  c = lax.axis_index("c")
  s = lax.axis_index("s")
  wid = s * 2 + c
  base = wid * _ROWS

  pltpu.sync_copy(pred_hbm.at[pl.ds(base * _NUM_BINS, _ROWS * _NUM_BINS)],
                  pred_v.at[pl.ds(0, _ROWS * _NUM_BINS)])
  pltpu.sync_copy(mz_hbm.at[pl.ds(base * _T, _ROWS * _T)],
                  mz_v.at[pl.ds(0, _ROWS * _T)])
  pltpu.sync_copy(it_hbm.at[pl.ds(base * _T, _ROWS * _T)],
                  it_v.at[pl.ds(0, _ROWS * _T)])
  pltpu.sync_copy(mk_hbm.at[pl.ds(base * _T, _ROWS * _T)],
                  mk_v.at[pl.ds(0, _ROWS * _T)])

  zeros = jnp.zeros((_LANES,), jnp.float32)
  lane = lax.iota(jnp.int32, _LANES)
  tail_mask = lane < _T_TAIL

  def zero_hist(j, carry):
    hist_v[pl.ds(j * _LANES, _LANES)] = zeros
    return carry

  lax.fori_loop(0, _NB_PAD // _LANES, zero_hist, 0)

  def row_body(r, carry):
    pbase = r * _NUM_BINS
    tbase = r * _T
    psplat = jnp.full((_LANES,), pbase, jnp.int32)

    dot = jnp.zeros((_LANES,), jnp.float32)
    for j in range(_TG):
      off = tbase + j * _LANES
      mz = mz_v[pl.ds(off, _LANES)]
      w = it_v[pl.ds(off, _LANES)] * mk_v[pl.ds(off, _LANES)]
      if j == _TG - 1:
        w = jnp.where(tail_mask, w, 0.0)
      idx = jnp.clip((mz * _MAX_MZ).astype(jnp.int32), 0, _NUM_BINS - 1)
      idx_v[pl.ds(j * _LANES, _LANES)] = idx
      w_v[pl.ds(j * _LANES, _LANES)] = w
      g = plsc.load_gather(pred_v, [psplat + idx])
      plsc.addupdate_scatter(hist_v, [idx], w)
      dot = dot + g * w

    tn2 = jnp.zeros((_LANES,), jnp.float32)
    for j in range(_TG):
      idx = idx_v[pl.ds(j * _LANES, _LANES)]
      w = w_v[pl.ds(j * _LANES, _LANES)]
      hv = plsc.load_gather(hist_v, [idx])
      tn2 = tn2 + hv * w

    for j in range(_TG):
      idx = idx_v[pl.ds(j * _LANES, _LANES)]
      plsc.store_scatter(hist_v, [idx], zeros)

    dot_v[pl.ds(r * _LANES, _LANES)] = dot
    tn2_v[pl.ds(r * _LANES, _LANES)] = tn2
    return carry

  lax.fori_loop(0, _ROWS, row_body, 0)

  pltpu.sync_copy(dot_v, dot_hbm.at[pl.ds(base * _LANES, _ROWS * _LANES)])
  pltpu.sync_copy(tn2_v, tn2_hbm.at[pl.ds(base * _LANES, _ROWS * _LANES)])


def _pn2_body(pred_ref, out_ref):
  x = pred_ref[...]
  out_ref[...] = jnp.sum(x * x, axis=1, keepdims=True)


def _combine_body(dot_ref, tn2_ref, pn2_ref, out_ref):
  dot = jnp.sum(dot_ref[...], axis=1)
  tnorm = jnp.sqrt(jnp.sum(tn2_ref[...], axis=1))
  pnorm = jnp.sqrt(pn2_ref[...][:, 0])
  num = dot / ((pnorm + 1e-8) * (tnorm + 1e-8))
  na = jnp.maximum(pnorm / (pnorm + 1e-8), 1e-8)
  nb = jnp.maximum(tnorm / (tnorm + 1e-8), 1e-8)
  cos = num / (na * nb)
  out_ref[...] = jnp.broadcast_to(1.0 - jnp.mean(cos), (1, 1))


def kernel(pred_binned, target_mz, target_intensity, target_mask):
  sc = pl.kernel(
      _sc_body,
      mesh=plsc.VectorSubcoreMesh(core_axis_name="c", subcore_axis_name="s"),
      compiler_params=pltpu.CompilerParams(
          use_tc_tiling_on_sc=False, needs_layout_passes=False),
      out_type=(
          jax.ShapeDtypeStruct((_B * _LANES,), jnp.float32),
          jax.ShapeDtypeStruct((_B * _LANES,), jnp.float32),
      ),
      scratch_types=[
          pltpu.VMEM((_ROWS * _NUM_BINS + _LANES,), jnp.float32),
          pltpu.VMEM((_ROWS * _T + _LANES,), jnp.float32),
          pltpu.VMEM((_ROWS * _T + _LANES,), jnp.float32),
          pltpu.VMEM((_ROWS * _T + _LANES,), jnp.float32),
          pltpu.VMEM((_NB_PAD,), jnp.float32),
          pltpu.VMEM((_TG * _LANES,), jnp.int32),
          pltpu.VMEM((_TG * _LANES,), jnp.float32),
          pltpu.VMEM((_ROWS * _LANES,), jnp.float32),
          pltpu.VMEM((_ROWS * _LANES,), jnp.float32),
      ],
  )
  dot, tn2 = sc(pred_binned.reshape(-1), target_mz.reshape(-1),
                target_intensity.reshape(-1), target_mask.reshape(-1))

  pn2 = pl.pallas_call(
      _pn2_body,
      grid=(8,),
      in_specs=[pl.BlockSpec((_B // 8, _NUM_BINS), lambda i: (i, 0))],
      out_specs=pl.BlockSpec((_B // 8, 1), lambda i: (i, 0)),
      out_shape=jax.ShapeDtypeStruct((_B, 1), jnp.float32),
  )(pred_binned)

  out = pl.pallas_call(
      _combine_body,
      out_shape=jax.ShapeDtypeStruct((1, 1), jnp.float32),
  )(dot.reshape(_B, _LANES), tn2.reshape(_B, _LANES), pn2)
  return out.reshape(())
